# baseline (speedup 1.0000x reference)
import jax
import jax.numpy as jnp
from jax.experimental import pallas as pl
from jax.experimental.pallas import tpu as pltpu

_B = 1024
_N = 1024
_D = 64
_S = 8
_COMMIT = 0.25


def _vq_body(z_ref, cb_ref, qst_ref, idx_ref, loss_ref,
             cbm2_s, csq_s, cbt_s):
    i = pl.program_id(0)

    @pl.when(i == 0)
    def _():
        cb = cb_ref[...]
        cbm2_s[...] = -2.0 * cb
        csq_s[...] = jnp.sum(cb * cb, axis=1, keepdims=True)
        cbt_s[...] = jnp.transpose(cb)

    xt = z_ref[0]
    csq = csq_s[...]
    zc2 = jnp.dot(cbm2_s[...], xt,
                  preferred_element_type=jnp.float32)
    w = jnp.sum(xt * xt, axis=0, keepdims=True)
    wb = jnp.broadcast_to(w, (_S, _B))
    si = jax.lax.broadcasted_iota(jnp.int32, (_S, _B), 0)
    ch = 4
    per = (_N // _S) // ch
    items = []
    for c in range(ch):
        v = ixv = None
        for k in range(c * per, (c + 1) * per):
            sl = slice(k * _S, (k + 1) * _S)
            dk = (wb + csq[sl, :]) + zc2[sl, :]
            ik = si + (k * _S)
            if v is None:
                v, ixv = dk, ik
            else:
                m = v <= dk
                v = jnp.where(m, v, dk)
                ixv = jnp.where(m, ixv, ik)
        items.append((v, ixv))
    while len(items) > 1:
        nxt = []
        for j in range(0, len(items), 2):
            va, ia = items[j]
            vb, ib = items[j + 1]
            m = va <= vb
            nxt.append((jnp.where(m, va, vb), jnp.where(m, ia, ib)))
        items = nxt
    v, ix = items[0]
    dmin = jnp.min(v, axis=0, keepdims=True)
    idx = jnp.min(jnp.where(v == dmin, ix, _N), axis=0, keepdims=True)
    oh = (jax.lax.broadcasted_iota(jnp.int32, (_N, _B), 0)
          == idx).astype(jnp.float32)
    q = jnp.dot(cbt_s[...], oh,
                preferred_element_type=jnp.float32)
    d = q - xt
    qst_ref[0] = xt + d
    idx_ref[0] = idx
    loss_ref[0] = jnp.sum(jnp.sum(d * d, axis=1, keepdims=True), axis=0,
                          keepdims=True)


def kernel(z, codebook):
    b, dim, t = z.shape
    n = b * t
    tpb = t // _B
    qst, idx, lacc = pl.pallas_call(
        _vq_body,
        grid=(n // _B,),
        in_specs=[
            pl.BlockSpec((1, dim, _B), lambda i: (i // tpb, 0, i % tpb)),
            pl.BlockSpec((_N, dim), lambda i: (0, 0)),
        ],
        out_specs=[
            pl.BlockSpec((1, dim, _B), lambda i: (i // tpb, 0, i % tpb)),
            pl.BlockSpec((1, 1, _B), lambda i: (i // tpb, 0, i % tpb)),
            pl.BlockSpec((1, 1, 1), lambda i: (i, 0, 0)),
        ],
        out_shape=[
            jax.ShapeDtypeStruct((b, dim, t), jnp.float32),
            jax.ShapeDtypeStruct((b, 1, t), jnp.int32),
            jax.ShapeDtypeStruct((n // _B, 1, 1), jnp.float32),
        ],
        scratch_shapes=[
            pltpu.VMEM((_N, _D), jnp.float32),
            pltpu.VMEM((_N, 1), jnp.float32),
            pltpu.VMEM((_D, _N), jnp.float32),
        ],
    )(z, codebook)
    ls = jnp.sum(lacc) / (b * dim * t)
    loss = ls + _COMMIT * ls
    return (qst, loss, idx.reshape(n, 1))


# speedup vs baseline: 1.0496x; 1.0496x over previous
import jax
import jax.numpy as jnp
from jax.experimental import pallas as pl

_B = 1024
_N = 1024
_D = 64
_S = 8
_COMMIT = 0.25


def _vq_body(z_ref, cb_ref, qst_ref, idx_ref, loss_ref):
    i = pl.program_id(0)
    xt = z_ref[0]
    cb = cb_ref[...]
    cbm2 = -2.0 * cb
    csq = jnp.sum(cb * cb, axis=1, keepdims=True)
    zc2 = jnp.dot(cbm2, xt, preferred_element_type=jnp.float32)
    w = jnp.sum(xt * xt, axis=0, keepdims=True)
    wb = jnp.broadcast_to(w, (_S, _B))
    si = jax.lax.broadcasted_iota(jnp.int32, (_S, _B), 0)
    ch = 4
    per = (_N // _S) // ch
    items = []
    for c in range(ch):
        v = ixv = None
        for k in range(c * per, (c + 1) * per):
            sl = slice(k * _S, (k + 1) * _S)
            dk = (wb + csq[sl, :]) + zc2[sl, :]
            ik = si + (k * _S)
            if v is None:
                v, ixv = dk, ik
            else:
                m = v <= dk
                v = jnp.where(m, v, dk)
                ixv = jnp.where(m, ixv, ik)
        items.append((v, ixv))
    while len(items) > 1:
        nxt = []
        for j in range(0, len(items), 2):
            va, ia = items[j]
            vb, ib = items[j + 1]
            m = va <= vb
            nxt.append((jnp.where(m, va, vb), jnp.where(m, ia, ib)))
        items = nxt
    v, ix = items[0]
    dmin = jnp.min(v, axis=0, keepdims=True)
    idx = jnp.min(jnp.where(v == dmin, ix, _N), axis=0, keepdims=True)
    oh = (jax.lax.broadcasted_iota(jnp.int32, (_N, _B), 0)
          == idx).astype(jnp.float32)
    q = jnp.dot(jnp.transpose(cb), oh,
                preferred_element_type=jnp.float32)
    d = q - xt
    qst_ref[0] = xt + d
    idx_ref[0] = idx
    part = jnp.sum(jnp.sum(d * d, axis=1, keepdims=True), axis=0,
                   keepdims=True)

    @pl.when(i == 0)
    def _():
        loss_ref[...] = jnp.zeros_like(loss_ref)

    loss_ref[...] += part


def kernel(z, codebook):
    b, dim, t = z.shape
    n = b * t
    tpb = t // _B
    qst, idx, lacc = pl.pallas_call(
        _vq_body,
        grid=(n // _B,),
        in_specs=[
            pl.BlockSpec((1, dim, _B), lambda i: (i // tpb, 0, i % tpb)),
            pl.BlockSpec((_N, dim), lambda i: (0, 0)),
        ],
        out_specs=[
            pl.BlockSpec((1, dim, _B), lambda i: (i // tpb, 0, i % tpb)),
            pl.BlockSpec((1, 1, _B), lambda i: (i // tpb, 0, i % tpb)),
            pl.BlockSpec((1, 1), lambda i: (0, 0)),
        ],
        out_shape=[
            jax.ShapeDtypeStruct((b, dim, t), jnp.float32),
            jax.ShapeDtypeStruct((b, 1, t), jnp.int32),
            jax.ShapeDtypeStruct((1, 1), jnp.float32),
        ],
    )(z, codebook)
    ls = lacc[0, 0] / (b * dim * t)
    loss = ls + _COMMIT * ls
    return (qst, loss, idx.reshape(n, 1))


# speedup vs baseline: 1.1168x; 1.0640x over previous
import jax
import jax.numpy as jnp
from jax.experimental import pallas as pl

_B = 1024
_N = 1024
_D = 64
_S = 8
_COMMIT = 0.25


def _vq_body(z_ref, cb_ref, qst_ref, idx_ref, loss_ref):
    i = pl.program_id(0)
    xt = z_ref[0]
    cb = cb_ref[...]
    cbm2 = -2.0 * cb
    csq = jnp.sum(cb * cb, axis=1, keepdims=True)
    zc2 = jnp.dot(cbm2, xt, preferred_element_type=jnp.float32)
    w = jnp.sum(xt * xt, axis=0, keepdims=True)
    wb = jnp.broadcast_to(w, (_S, _B))
    si = jax.lax.broadcasted_iota(jnp.int32, (_S, _B), 0)
    ch = 2
    per = (_N // _S) // ch
    items = []
    for c in range(ch):
        v = ixv = None
        for k in range(c * per, (c + 1) * per):
            sl = slice(k * _S, (k + 1) * _S)
            dk = (wb + csq[sl, :]) + zc2[sl, :]
            ik = si + (k * _S)
            if v is None:
                v, ixv = dk, ik
            else:
                m = v <= dk
                v = jnp.where(m, v, dk)
                ixv = jnp.where(m, ixv, ik)
        items.append((v, ixv))
    while len(items) > 1:
        nxt = []
        for j in range(0, len(items), 2):
            va, ia = items[j]
            vb, ib = items[j + 1]
            m = va <= vb
            nxt.append((jnp.where(m, va, vb), jnp.where(m, ia, ib)))
        items = nxt
    v, ix = items[0]
    dmin = jnp.min(v, axis=0, keepdims=True)
    idx = jnp.min(jnp.where(v == dmin, ix, _N), axis=0, keepdims=True)
    oh = (jax.lax.broadcasted_iota(jnp.int32, (_N, _B), 0)
          == idx).astype(jnp.float32)
    q = jnp.dot(jnp.transpose(cb), oh,
                preferred_element_type=jnp.float32)
    d = q - xt
    qst_ref[0] = xt + d
    idx_ref[0] = idx
    part = jnp.sum(jnp.sum(d * d, axis=1, keepdims=True), axis=0,
                   keepdims=True)

    @pl.when(i == 0)
    def _():
        loss_ref[...] = jnp.zeros_like(loss_ref)

    acc = loss_ref[...] + part
    ls = acc / (pl.num_programs(0) * _B * _D)
    loss_ref[...] = jnp.where(i == pl.num_programs(0) - 1,
                              ls + _COMMIT * ls, acc)


def kernel(z, codebook):
    b, dim, t = z.shape
    n = b * t
    tpb = t // _B
    qst, idx, lacc = pl.pallas_call(
        _vq_body,
        grid=(n // _B,),
        in_specs=[
            pl.BlockSpec((1, dim, _B), lambda i: (i // tpb, 0, i % tpb)),
            pl.BlockSpec((_N, dim), lambda i: (0, 0)),
        ],
        out_specs=[
            pl.BlockSpec((1, dim, _B), lambda i: (i // tpb, 0, i % tpb)),
            pl.BlockSpec((1, 1, _B), lambda i: (i // tpb, 0, i % tpb)),
            pl.BlockSpec((1, 1), lambda i: (0, 0)),
        ],
        out_shape=[
            jax.ShapeDtypeStruct((b, dim, t), jnp.float32),
            jax.ShapeDtypeStruct((b, 1, t), jnp.int32),
            jax.ShapeDtypeStruct((1, 1), jnp.float32),
        ],
    )(z, codebook)
    return (qst, lacc[0, 0], idx.reshape(n, 1))


# speedup vs baseline: 1.1202x; 1.0031x over previous
import jax
import jax.numpy as jnp
from jax.experimental import pallas as pl

_B = 1024
_N = 1024
_D = 64
_S = 8
_COMMIT = 0.25


def _vq_body(z_ref, cb_ref, qst_ref, idx_ref, loss_ref):
    i = pl.program_id(0)
    xt = z_ref[0]
    cb = cb_ref[...]
    cbm2 = -2.0 * cb
    csq = jnp.sum(cb * cb, axis=1, keepdims=True)
    zc2 = jnp.dot(cbm2, xt, preferred_element_type=jnp.float32)
    w = jnp.sum(xt * xt, axis=0, keepdims=True)
    wb = jnp.broadcast_to(w, (_S, _B))
    si = jax.lax.broadcasted_iota(jnp.int32, (_S, _B), 0)
    ch = 1
    per = (_N // _S) // ch
    items = []
    for c in range(ch):
        v = ixv = None
        for k in range(c * per, (c + 1) * per):
            sl = slice(k * _S, (k + 1) * _S)
            dk = (wb + csq[sl, :]) + zc2[sl, :]
            ik = si + (k * _S)
            if v is None:
                v, ixv = dk, ik
            else:
                m = v <= dk
                v = jnp.where(m, v, dk)
                ixv = jnp.where(m, ixv, ik)
        items.append((v, ixv))
    while len(items) > 1:
        nxt = []
        for j in range(0, len(items), 2):
            va, ia = items[j]
            vb, ib = items[j + 1]
            m = va <= vb
            nxt.append((jnp.where(m, va, vb), jnp.where(m, ia, ib)))
        items = nxt
    v, ix = items[0]
    dmin = jnp.min(v, axis=0, keepdims=True)
    idx = jnp.min(jnp.where(v == dmin, ix, _N), axis=0, keepdims=True)
    oh = (jax.lax.broadcasted_iota(jnp.int32, (_N, _B), 0)
          == idx).astype(jnp.float32)
    q = jnp.dot(jnp.transpose(cb), oh,
                preferred_element_type=jnp.float32)
    d = q - xt
    qst_ref[0] = xt + d
    idx_ref[0] = idx
    part = jnp.sum(jnp.sum(d * d, axis=1, keepdims=True), axis=0,
                   keepdims=True)

    @pl.when(i == 0)
    def _():
        loss_ref[...] = jnp.zeros_like(loss_ref)

    acc = loss_ref[...] + part
    ls = acc / (pl.num_programs(0) * _B * _D)
    loss_ref[...] = jnp.where(i == pl.num_programs(0) - 1,
                              ls + _COMMIT * ls, acc)


def kernel(z, codebook):
    b, dim, t = z.shape
    n = b * t
    tpb = t // _B
    qst, idx, lacc = pl.pallas_call(
        _vq_body,
        grid=(n // _B,),
        in_specs=[
            pl.BlockSpec((1, dim, _B), lambda i: (i // tpb, 0, i % tpb)),
            pl.BlockSpec((_N, dim), lambda i: (0, 0)),
        ],
        out_specs=[
            pl.BlockSpec((1, dim, _B), lambda i: (i // tpb, 0, i % tpb)),
            pl.BlockSpec((1, 1, _B), lambda i: (i // tpb, 0, i % tpb)),
            pl.BlockSpec((1, 1), lambda i: (0, 0)),
        ],
        out_shape=[
            jax.ShapeDtypeStruct((b, dim, t), jnp.float32),
            jax.ShapeDtypeStruct((b, 1, t), jnp.int32),
            jax.ShapeDtypeStruct((1, 1), jnp.float32),
        ],
    )(z, codebook)
    return (qst, lacc[0, 0], idx.reshape(n, 1))
